# baseline (speedup 1.0000x reference)
import functools

import jax
import jax.numpy as jnp
from jax import lax
from jax.experimental import pallas as pl
from jax.experimental.pallas import tpu as pltpu
from jax.experimental.pallas import tpu_sc as plsc

VOCAB = 100000
MAXLEN = 200
DIM = 32
BATCH = 4096

NUM_CORES = 2
NUM_SUBCORES = 16
NUM_WORKERS = NUM_CORES * NUM_SUBCORES

LANES = 128
SUBL = 8
VTILES = (VOCAB + LANES - 1) // LANES
VPAD = VTILES * LANES
DTILES = DIM // SUBL

BBLK = 32
NBLK = BATCH // BBLK
VEC = 16


def _sc_embed(tphys, idx_flat, pos16, bpat):
    mesh = plsc.VectorSubcoreMesh(core_axis_name="c", subcore_axis_name="s")

    @functools.partial(
        pl.kernel,
        mesh=mesh,
        compiler_params=pltpu.CompilerParams(
            use_tc_tiling_on_sc=False, needs_layout_passes=False
        ),
        out_type=jax.ShapeDtypeStruct((BATCH * MAXLEN, DIM), jnp.float32),
        scratch_types=[
            pltpu.VMEM((VTILES, LANES), jnp.float32),
            pltpu.VMEM((BBLK * MAXLEN,), jnp.int32),
            pltpu.VMEM((BBLK * MAXLEN,), jnp.int32),
            pltpu.VMEM((MAXLEN, BBLK), jnp.float32),
            pltpu.VMEM((MAXLEN, BBLK), jnp.float32),
            pltpu.VMEM((MAXLEN * VEC,), jnp.float32),
            pltpu.VMEM((BBLK,), jnp.int32),
            pltpu.VMEM((MAXLEN + VEC,), jnp.int32),
            pltpu.VMEM((MAXLEN + VEC,), jnp.int32),
            pltpu.VMEM((MAXLEN + VEC,), jnp.int32),
            pltpu.SemaphoreType.DMA,
            pltpu.SemaphoreType.DMA,
            pltpu.SemaphoreType.DMA,
            pltpu.SemaphoreType.DMA,
        ],
    )
    def k(tbl_hbm, idx_hbm, pos_hbm, bpat_hbm, lconst_hbm, out_hbm,
          tbl_v, xa, xb, sa, sb, pos_v, bpat_v, ra, rb, lconst_v,
          in_a, in_b, out_a, out_b):
        d = lax.axis_index("s") * NUM_CORES + lax.axis_index("c")
        dt = d // SUBL
        ds = d % SUBL

        pltpu.sync_copy(tbl_hbm.at[dt, :, ds, :], tbl_v)
        pltpu.sync_copy(pos_hbm.at[pl.ds(d * MAXLEN * VEC, MAXLEN * VEC)],
                        pos_v)
        pltpu.sync_copy(bpat_hbm, bpat_v)
        pltpu.sync_copy(lconst_hbm, lconst_v)
        b_lo = bpat_v[pl.ds(0, VEC)]
        b_hi = bpat_v[pl.ds(VEC, VEC)]
        zero16 = jnp.zeros((VEC,), jnp.int32)
        col0 = lax.iota(jnp.int32, VEC)

        def start_in(blk, xv, sem):
            pltpu.async_copy(
                idx_hbm.at[pl.ds(blk * (BBLK * MAXLEN), BBLK * MAXLEN)],
                xv, sem)

        def wait_in(blk, xv, sem):
            pltpu.make_async_copy(
                idx_hbm.at[pl.ds(blk * (BBLK * MAXLEN), BBLK * MAXLEN)],
                xv, sem).wait()

        def fill_ridx(blk, rv):
            bt = blk // (LANES // BBLK)
            q = blk % (LANES // BBLK)
            rbase = ((dt * (BATCH // LANES) + bt) * SUBL + ds) * 4 + q
            for i in range((MAXLEN + VEC) // VEC):
                rv.at[pl.ds(i * VEC, VEC)][...] = (
                    lconst_v[pl.ds(i * VEC, VEC)] + rbase
                )

        def start_out(blk, sv, rv, sem):
            fill_ridx(blk, rv)
            pltpu.async_copy(sv, out_hbm.at[rv.at[pl.ds(0, MAXLEN)]], sem)

        def wait_out(sv, rv, sem):
            pltpu.make_async_copy(
                sv, out_hbm.at[rv.at[pl.ds(0, MAXLEN)]], sem).wait()

        def compute(xv, sv):
            @pl.loop(0, MAXLEN, step=4)
            def _(l0):
                for dl in range(4):
                    l = l0 + dl
                    pv = pos_v[pl.ds(l * VEC, VEC)]
                    lvec = zero16 + l
                    for h, bvec in ((0, b_lo), (1, b_hi)):
                        xi = plsc.load_gather(xv, [bvec + l])
                        tv = plsc.load_gather(tbl_v, [zero16, xi])
                        plsc.store_scatter(
                            sv, [lvec, col0 + h * VEC], tv + pv)

        start_in(0, xa, in_a)

        @pl.loop(0, NBLK, step=2)
        def _(k0):
            start_in(k0 + 1, xb, in_b)
            wait_in(k0, xa, in_a)

            @pl.when(k0 >= 2)
            def _():
                wait_out(sa, ra, out_a)

            compute(xa, sa)
            start_out(k0, sa, ra, out_a)

            @pl.when(k0 + 2 < NBLK)
            def _():
                start_in(k0 + 2, xa, in_a)

            wait_in(k0 + 1, xb, in_b)

            @pl.when(k0 >= 2)
            def _():
                wait_out(sb, rb, out_b)

            compute(xb, sb)
            start_out(k0 + 1, sb, rb, out_b)

        wait_out(sa, ra, out_a)
        wait_out(sb, rb, out_b)

    lconst = jnp.arange(MAXLEN + VEC, dtype=jnp.int32) * (DIM * LANES)
    return k(tphys, idx_flat, pos16, bpat, lconst)


def kernel(x, token_table, pos_table):
    tphys = jnp.pad(token_table.astype(jnp.float32), ((0, VPAD - VOCAB), (0, 0)))
    tphys = tphys.reshape(VTILES, LANES, DTILES, SUBL).transpose(2, 0, 3, 1)
    idx_flat = x.reshape(BATCH * MAXLEN).astype(jnp.int32)
    pos16 = jnp.repeat(
        pos_table.astype(jnp.float32).T[:, :, None], VEC, axis=2
    ).reshape(DIM * MAXLEN * VEC)
    bpat = jnp.arange(BBLK, dtype=jnp.int32) * MAXLEN

    out2 = _sc_embed(tphys, idx_flat, pos16, bpat)
    out5 = out2.reshape(MAXLEN, DTILES, BATCH // LANES, SUBL, LANES)
    return out5.transpose(2, 4, 0, 1, 3).reshape(BATCH, MAXLEN, DIM)


# speedup vs baseline: 3.2096x; 3.2096x over previous
import functools

import jax
import jax.numpy as jnp
from jax import lax
from jax.experimental import pallas as pl
from jax.experimental.pallas import tpu as pltpu
from jax.experimental.pallas import tpu_sc as plsc

VOCAB = 100000
MAXLEN = 200
DIM = 32
BATCH = 4096

NUM_CORES = 2
NUM_SUBCORES = 16
NUM_WORKERS = NUM_CORES * NUM_SUBCORES

LANES = 128
SUBL = 8
VTILES = (VOCAB + LANES - 1) // LANES
VPAD = VTILES * LANES
DTILES = DIM // SUBL

BBLK = 32
NBLK = BATCH // BBLK
VEC = 16


def _sc_embed(tphys, idx_flat, pos16, bpat):
    mesh = plsc.VectorSubcoreMesh(core_axis_name="c", subcore_axis_name="s")

    @functools.partial(
        pl.kernel,
        mesh=mesh,
        compiler_params=pltpu.CompilerParams(
            use_tc_tiling_on_sc=False, needs_layout_passes=False
        ),
        out_type=jax.ShapeDtypeStruct((BATCH * MAXLEN, DIM), jnp.float32),
        scratch_types=[
            pltpu.VMEM((VTILES, LANES), jnp.float32),
            pltpu.VMEM((BBLK * MAXLEN,), jnp.int32),
            pltpu.VMEM((BBLK * MAXLEN,), jnp.int32),
            pltpu.VMEM((MAXLEN, BBLK), jnp.float32),
            pltpu.VMEM((MAXLEN, BBLK), jnp.float32),
            pltpu.VMEM((MAXLEN * VEC,), jnp.float32),
            pltpu.VMEM((BBLK,), jnp.int32),
            pltpu.VMEM((MAXLEN + VEC,), jnp.int32),
            pltpu.VMEM((MAXLEN + VEC,), jnp.int32),
            pltpu.VMEM((MAXLEN + VEC,), jnp.int32),
            pltpu.SemaphoreType.DMA,
            pltpu.SemaphoreType.DMA,
            pltpu.SemaphoreType.DMA,
            pltpu.SemaphoreType.DMA,
        ],
    )
    def k(tbl_hbm, idx_hbm, pos_hbm, bpat_hbm, lconst_hbm, out_hbm,
          tbl_v, xa, xb, sa, sb, pos_v, bpat_v, ra, rb, lconst_v,
          in_a, in_b, out_a, out_b):
        d = lax.axis_index("s") * NUM_CORES + lax.axis_index("c")
        dt = d // SUBL
        ds = d % SUBL

        pltpu.sync_copy(tbl_hbm.at[dt, :, ds, :], tbl_v)
        pltpu.sync_copy(pos_hbm.at[pl.ds(d * MAXLEN * VEC, MAXLEN * VEC)],
                        pos_v)
        pltpu.sync_copy(bpat_hbm, bpat_v)
        pltpu.sync_copy(lconst_hbm, lconst_v)
        b_lo = bpat_v[pl.ds(0, VEC)]
        b_hi = bpat_v[pl.ds(VEC, VEC)]
        zero16 = jnp.zeros((VEC,), jnp.int32)
        col0 = lax.iota(jnp.int32, VEC)

        def start_in(blk, xv, sem):
            pltpu.async_copy(
                idx_hbm.at[pl.ds(blk * (BBLK * MAXLEN), BBLK * MAXLEN)],
                xv, sem)

        def wait_in(blk, xv, sem):
            pltpu.make_async_copy(
                idx_hbm.at[pl.ds(blk * (BBLK * MAXLEN), BBLK * MAXLEN)],
                xv, sem).wait()

        def fill_ridx(blk, rv):
            bt = blk // (LANES // BBLK)
            q = blk % (LANES // BBLK)
            rbase = ((dt * (BATCH // LANES) + bt) * SUBL + ds) * 4 + q
            for i in range((MAXLEN + VEC) // VEC):
                rv.at[pl.ds(i * VEC, VEC)][...] = (
                    lconst_v[pl.ds(i * VEC, VEC)] + rbase
                )

        def start_out(blk, sv, rv, sem):
            fill_ridx(blk, rv)
            pltpu.async_copy(sv, out_hbm.at[rv.at[pl.ds(0, MAXLEN)]], sem)

        def wait_out(sv, rv, sem):
            pltpu.make_async_copy(
                sv, out_hbm.at[rv.at[pl.ds(0, MAXLEN)]], sem).wait()

        def compute(xv, sv):
            @plsc.parallel_loop(0, MAXLEN, unroll=4)
            def _(l):
                pv = pos_v[pl.ds(l * VEC, VEC)]
                lvec = zero16 + l
                for h, bvec in ((0, b_lo), (1, b_hi)):
                    xi = plsc.load_gather(xv, [bvec + l])
                    tv = plsc.load_gather(tbl_v, [zero16, xi])
                    plsc.store_scatter(
                        sv, [lvec, col0 + h * VEC], tv + pv)

        start_in(0, xa, in_a)

        @pl.loop(0, NBLK, step=2)
        def _(k0):
            start_in(k0 + 1, xb, in_b)
            wait_in(k0, xa, in_a)

            @pl.when(k0 >= 2)
            def _():
                wait_out(sa, ra, out_a)

            compute(xa, sa)
            start_out(k0, sa, ra, out_a)

            @pl.when(k0 + 2 < NBLK)
            def _():
                start_in(k0 + 2, xa, in_a)

            wait_in(k0 + 1, xb, in_b)

            @pl.when(k0 >= 2)
            def _():
                wait_out(sb, rb, out_b)

            compute(xb, sb)
            start_out(k0 + 1, sb, rb, out_b)

        wait_out(sa, ra, out_a)
        wait_out(sb, rb, out_b)

    lconst = jnp.arange(MAXLEN + VEC, dtype=jnp.int32) * (DIM * LANES)
    return k(tphys, idx_flat, pos16, bpat, lconst)


def kernel(x, token_table, pos_table):
    tphys = jnp.pad(token_table.astype(jnp.float32), ((0, VPAD - VOCAB), (0, 0)))
    tphys = tphys.reshape(VTILES, LANES, DTILES, SUBL).transpose(2, 0, 3, 1)
    idx_flat = x.reshape(BATCH * MAXLEN).astype(jnp.int32)
    pos16 = jnp.repeat(
        pos_table.astype(jnp.float32).T[:, :, None], VEC, axis=2
    ).reshape(DIM * MAXLEN * VEC)
    bpat = jnp.arange(BBLK, dtype=jnp.int32) * MAXLEN

    out2 = _sc_embed(tphys, idx_flat, pos16, bpat)
    out5 = out2.reshape(MAXLEN, DTILES, BATCH // LANES, SUBL, LANES)
    return out5.transpose(2, 4, 0, 1, 3).reshape(BATCH, MAXLEN, DIM)
